# baseline (speedup 1.0000x reference)
import jax
import jax.numpy as jnp
from jax.experimental import pallas as pl
from jax.experimental.pallas import tpu as pltpu

SQRT3 = 3 ** 0.5


def _quant_body(x_ref, o_ref):
    x = x_ref[...]
    lane = jax.lax.broadcasted_iota(jnp.int32, x.shape, dimension=1)
    odd = (lane & 1) == 1
    scale = jnp.where(odd, jnp.float32(SQRT3), jnp.float32(1.0))
    xs = x / scale
    y1 = jnp.round(xs) * scale
    y2 = (jnp.round(xs - 0.5) + 0.5) * scale
    d1 = (x - y1) * (x - y1)
    d2 = (x - y2) * (x - y2)
    p1 = jnp.where(odd, jnp.roll(d1, 1, axis=1), jnp.roll(d1, -1, axis=1))
    p2 = jnp.where(odd, jnp.roll(d2, 1, axis=1), jnp.roll(d2, -1, axis=1))
    s1 = jnp.sqrt(d1 + p1)
    s2 = jnp.sqrt(d2 + p2)
    o_ref[...] = jnp.where(s1 <= s2, y1, y2)


def kernel(x):
    n = x.shape[0]
    xf = x.reshape(n // 64, 128)
    rows = xf.shape[0]
    block = 2048
    out = pl.pallas_call(
        _quant_body,
        grid=(rows // block,),
        in_specs=[pl.BlockSpec((block, 128), lambda i: (i, 0))],
        out_specs=pl.BlockSpec((block, 128), lambda i: (i, 0)),
        out_shape=jax.ShapeDtypeStruct((rows, 128), jnp.float32),
    )(xf)
    return out.reshape(n, 2)


# speedup vs baseline: 139.0272x; 139.0272x over previous
import jax
import jax.numpy as jnp
from jax.experimental import pallas as pl
from jax.experimental.pallas import tpu as pltpu

SQRT3 = 3 ** 0.5


def _quant_body(x_ref, o_ref):
    x = x_ref[...]
    row = jax.lax.broadcasted_iota(jnp.int32, x.shape, dimension=0)
    odd = (row & 1) == 1
    scale = jnp.where(odd, jnp.float32(SQRT3), jnp.float32(1.0))
    xs = x / scale
    y1 = jnp.round(xs) * scale
    y2 = (jnp.round(xs - 0.5) + 0.5) * scale
    d1 = (x - y1) * (x - y1)
    d2 = (x - y2) * (x - y2)
    p1 = jnp.where(odd, jnp.roll(d1, 1, axis=0), jnp.roll(d1, -1, axis=0))
    p2 = jnp.where(odd, jnp.roll(d2, 1, axis=0), jnp.roll(d2, -1, axis=0))
    s1 = jnp.sqrt(d1 + p1)
    s2 = jnp.sqrt(d2 + p2)
    o_ref[...] = jnp.where(s1 <= s2, y1, y2)


def kernel(x):
    n = x.shape[0]
    xf = x.reshape(n // 128, 128, 2).transpose(0, 2, 1).reshape(n // 64, 128)
    rows = xf.shape[0]
    block = 2048
    out = pl.pallas_call(
        _quant_body,
        grid=(rows // block,),
        in_specs=[pl.BlockSpec((block, 128), lambda i: (i, 0))],
        out_specs=pl.BlockSpec((block, 128), lambda i: (i, 0)),
        out_shape=jax.ShapeDtypeStruct((rows, 128), jnp.float32),
    )(xf)
    return out.reshape(n // 128, 2, 128).transpose(0, 2, 1).reshape(n, 2)
